# baseline (speedup 1.0000x reference)
import jax
import jax.numpy as jnp
from jax import lax
from jax.experimental import pallas as pl
from jax.experimental.pallas import tpu as pltpu
from jax.experimental.pallas import tpu_sc as plsc

_INFO = plsc.get_sparse_core_info()
_NUM_WORKERS = _INFO.num_cores * _INFO.num_subcores

_NBUF = 3
_PF = 1
_CHUNK_ROWS = 32


def _copy_body(rows_per_worker, dim, w_hbm, out_hbm, buf, sems_in, sems_out):
    wid = lax.axis_index("s") * _INFO.num_cores + lax.axis_index("c")
    base = wid * rows_per_worker
    nch = rows_per_worker // _CHUNK_ROWS

    def in_copy(c, b):
        return pltpu.make_async_copy(
            w_hbm.at[pl.ds(base + c * _CHUNK_ROWS, _CHUNK_ROWS)],
            buf.at[b],
            sems_in.at[b],
        )

    def out_copy(c, b):
        return pltpu.make_async_copy(
            buf.at[b],
            out_hbm.at[pl.ds(base + c * _CHUNK_ROWS, _CHUNK_ROWS)],
            sems_out.at[b],
        )

    outs_unwaited = []
    for c in range(min(_PF, nch)):
        in_copy(c, c % _NBUF).start()
    for c in range(nch):
        b = c % _NBUF
        in_copy(c, b).wait()
        out_copy(c, b).start()
        outs_unwaited.append((c, b))
        nxt = c + _PF
        if nxt < nch:
            prev = nxt - _NBUF
            if prev >= 0:
                out_copy(prev, nxt % _NBUF).wait()
                outs_unwaited.remove((prev, nxt % _NBUF))
            in_copy(nxt, nxt % _NBUF).start()
    for c, b in outs_unwaited:
        out_copy(c, b).wait()


def kernel(x, emb_weight):
    seq = x.shape[1]
    dim = emb_weight.shape[1]
    rows_per_worker = seq // _NUM_WORKERS
    mesh = plsc.VectorSubcoreMesh(core_axis_name="c", subcore_axis_name="s")
    out = pl.kernel(
        lambda w, o, buf, si, so: _copy_body(
            rows_per_worker, dim, w, o, buf, si, so
        ),
        out_type=jax.ShapeDtypeStruct((seq, dim), emb_weight.dtype),
        mesh=mesh,
        scratch_types=[
            pltpu.VMEM((_NBUF, _CHUNK_ROWS, dim), jnp.float32),
            pltpu.SemaphoreType.DMA((_NBUF,)),
            pltpu.SemaphoreType.DMA((_NBUF,)),
        ],
    )(emb_weight)
    return out[None]


# speedup vs baseline: 1.8215x; 1.8215x over previous
import jax
import jax.numpy as jnp
from jax.experimental import pallas as pl
from jax.experimental.pallas import tpu as pltpu

_BLOCK_ROWS = 512


def _copy_body(w_ref, o_ref):
    o_ref[...] = w_ref[...]


def kernel(x, emb_weight):
    seq = x.shape[1]
    dim = emb_weight.shape[1]
    grid = (seq // _BLOCK_ROWS,)
    out = pl.pallas_call(
        _copy_body,
        grid=grid,
        in_specs=[pl.BlockSpec((_BLOCK_ROWS, dim), lambda i: (i, 0))],
        out_specs=pl.BlockSpec((_BLOCK_ROWS, dim), lambda i: (i, 0)),
        out_shape=jax.ShapeDtypeStruct((seq, dim), emb_weight.dtype),
    )(emb_weight)
    return out[None]


# speedup vs baseline: 2.1377x; 1.1736x over previous
import jax
import jax.numpy as jnp
from jax.experimental import pallas as pl
from jax.experimental.pallas import tpu as pltpu

_BLOCK_ROWS = 2048


def _copy_body(w_ref, o_ref):
    o_ref[...] = w_ref[...]


def kernel(x, emb_weight):
    seq = x.shape[1]
    dim = emb_weight.shape[1]
    grid = (seq // _BLOCK_ROWS,)
    out = pl.pallas_call(
        _copy_body,
        grid=grid,
        in_specs=[pl.BlockSpec((_BLOCK_ROWS, dim), lambda i: (i, 0))],
        out_specs=pl.BlockSpec((_BLOCK_ROWS, dim), lambda i: (i, 0)),
        out_shape=jax.ShapeDtypeStruct((seq, dim), emb_weight.dtype),
    )(emb_weight)
    return out[None]
